# initial kernel scaffold (unmeasured)
import jax
import jax.numpy as jnp
from jax import lax
from jax.experimental import pallas as pl
from jax.experimental.pallas import tpu as pltpu

M = 2048
CHUNK = 2048


def kernel(dy, W):
    r = lax.axis_index("y") * 2 + lax.axis_index("z")
    a = lax.dynamic_slice_in_dim(dy, r * CHUNK, CHUNK, axis=1).astype(jnp.bfloat16)
    b = lax.dynamic_slice_in_dim(W, r * CHUNK, CHUNK, axis=1).astype(jnp.bfloat16)

    def body(a_ref, b_ref, out_ref, recv_ref, send_sems, recv_sems):
        out_ref[...] = lax.dot_general(
            a_ref[...],
            b_ref[...],
            dimension_numbers=(((1,), (1,)), ((), ())),
            preferred_element_type=jnp.float32,
        )

        mx = lax.axis_index("x")
        my = lax.axis_index("y")
        mz = lax.axis_index("z")
        neighbors = [
            (1 - mx, my, mz),
            (mx, 1 - my, mz),
            (mx, my, 1 - mz),
        ]

        barrier_sem = pltpu.get_barrier_semaphore()
        for nbr in neighbors:
            pl.semaphore_signal(
                barrier_sem, inc=1, device_id=nbr,
                device_id_type=pl.DeviceIdType.MESH,
            )
        pl.semaphore_wait(barrier_sem, 3)

        for k, nbr in enumerate(neighbors):
            rdma = pltpu.make_async_remote_copy(
                src_ref=out_ref,
                dst_ref=recv_ref.at[k],
                send_sem=send_sems.at[k],
                recv_sem=recv_sems.at[k],
                device_id=nbr,
                device_id_type=pl.DeviceIdType.MESH,
            )
            rdma.start()
            rdma.wait()
            out_ref[...] += recv_ref[k]

    return pl.pallas_call(
        body,
        out_shape=jax.ShapeDtypeStruct((M, M), jnp.float32),
        in_specs=[
            pl.BlockSpec(memory_space=pltpu.VMEM),
            pl.BlockSpec(memory_space=pltpu.VMEM),
        ],
        out_specs=pl.BlockSpec(memory_space=pltpu.VMEM),
        scratch_shapes=[
            pltpu.VMEM((3, M, M), jnp.float32),
            pltpu.SemaphoreType.DMA((3,)),
            pltpu.SemaphoreType.DMA((3,)),
        ],
        compiler_params=pltpu.CompilerParams(collective_id=0),
    )(a, b)


# baseline (device time: 388888 ns/iter reference)
import jax
import jax.numpy as jnp
from jax import lax
from jax.experimental import pallas as pl
from jax.experimental.pallas import tpu as pltpu

M = 2048
CHUNK = 2048


def kernel(dy, W):
    r = lax.axis_index("y") * 2 + lax.axis_index("z")
    a = lax.dynamic_slice_in_dim(dy, r * CHUNK, CHUNK, axis=1).astype(jnp.bfloat16)
    b = lax.dynamic_slice_in_dim(W, r * CHUNK, CHUNK, axis=1).astype(jnp.bfloat16)

    def body(a_ref, b_ref, out_ref, rs0, rs1, rs2, send_sems, recv_sems):
        out_ref[...] = lax.dot_general(
            a_ref[...],
            b_ref[...],
            dimension_numbers=(((1,), (1,)), ((), ())),
            preferred_element_type=jnp.float32,
        )

        mx = lax.axis_index("x")
        my = lax.axis_index("y")
        mz = lax.axis_index("z")
        x_nbr = (1 - mx, my, mz)
        y_nbr = (mx, 1 - my, mz)
        z_nbr = (mx, my, 1 - mz)

        barrier_sem = pltpu.get_barrier_semaphore()
        for nbr in (x_nbr, y_nbr, z_nbr):
            pl.semaphore_signal(
                barrier_sem, inc=1, device_id=nbr,
                device_id_type=pl.DeviceIdType.MESH,
            )
        pl.semaphore_wait(barrier_sem, 3)

        def exchange(k, src, dst, nbr):
            rdma = pltpu.make_async_remote_copy(
                src_ref=src,
                dst_ref=dst,
                send_sem=send_sems.at[k],
                recv_sem=recv_sems.at[k],
                device_id=nbr,
                device_id_type=pl.DeviceIdType.MESH,
            )
            rdma.start()
            rdma.wait()

        exchange(0, out_ref.at[pl.ds((1 - mx) * 1024, 1024), :], rs0, x_nbr)
        out_ref[pl.ds(mx * 1024, 1024), :] += rs0[...]

        exchange(1, out_ref.at[pl.ds(mx * 1024 + (1 - my) * 512, 512), :], rs1, y_nbr)
        out_ref[pl.ds(mx * 1024 + my * 512, 512), :] += rs1[...]

        base = mx * 1024 + my * 512 + mz * 256
        exchange(2, out_ref.at[pl.ds(mx * 1024 + my * 512 + (1 - mz) * 256, 256), :], rs2, z_nbr)
        out_ref[pl.ds(base, 256), :] += rs2[...]

        exchange(3, out_ref.at[pl.ds(base, 256), :],
                 out_ref.at[pl.ds(base, 256), :], z_nbr)
        exchange(4, out_ref.at[pl.ds(mx * 1024 + my * 512, 512), :],
                 out_ref.at[pl.ds(mx * 1024 + my * 512, 512), :], y_nbr)
        exchange(5, out_ref.at[pl.ds(mx * 1024, 1024), :],
                 out_ref.at[pl.ds(mx * 1024, 1024), :], x_nbr)

    return pl.pallas_call(
        body,
        out_shape=jax.ShapeDtypeStruct((M, M), jnp.float32),
        in_specs=[
            pl.BlockSpec(memory_space=pltpu.VMEM),
            pl.BlockSpec(memory_space=pltpu.VMEM),
        ],
        out_specs=pl.BlockSpec(memory_space=pltpu.VMEM),
        scratch_shapes=[
            pltpu.VMEM((1024, M), jnp.float32),
            pltpu.VMEM((512, M), jnp.float32),
            pltpu.VMEM((256, M), jnp.float32),
            pltpu.SemaphoreType.DMA((6,)),
            pltpu.SemaphoreType.DMA((6,)),
        ],
        compiler_params=pltpu.CompilerParams(
            collective_id=0,
            vmem_limit_bytes=61 * 1024 * 1024,
        ),
    )(a, b)


# device time: 137180 ns/iter; 2.8349x vs baseline; 2.8349x over previous
import jax
import jax.numpy as jnp
from jax import lax
from jax.experimental import pallas as pl
from jax.experimental.pallas import tpu as pltpu

M = 2048
CHUNK = 2048

AXES = ("x", "y", "z")
GROUPS = ((0, 768), (768, 640), (1408, 640))
ORDERS = ((0, 1, 2), (1, 2, 0), (2, 0, 1))


def kernel(dy, W):
    r = lax.axis_index("y") * 2 + lax.axis_index("z")
    a = lax.dynamic_slice_in_dim(dy, r * CHUNK, CHUNK, axis=1).astype(jnp.bfloat16)
    b = lax.dynamic_slice_in_dim(W, r * CHUNK, CHUNK, axis=1).astype(jnp.bfloat16)

    recv_off = []
    recv_rows = []
    for _, rows in GROUPS:
        sizes = [rows // 2, rows // 4, rows // 8, rows // 8, rows // 4, rows // 2]
        offs = []
        o = 0
        for s in sizes:
            offs.append(o)
            o += s
        recv_off.append(offs)
        recv_rows.append(o)

    def body(a_ref, b_ref, out_ref, rv0, rv1, rv2, st0, st1, st2,
             send_sems, recv_sems):
        for i in range(4):
            out_ref[i * 512:(i + 1) * 512, :] = lax.dot_general(
                a_ref[i * 512:(i + 1) * 512, :],
                b_ref[...],
                dimension_numbers=(((1,), (1,)), ((), ())),
                preferred_element_type=jnp.float32,
            )

        m = [lax.axis_index(ax) for ax in AXES]

        def nbr(ax):
            t = list(m)
            t[ax] = 1 - t[ax]
            return tuple(t)

        barrier_sem = pltpu.get_barrier_semaphore()
        for ax in range(3):
            pl.semaphore_signal(
                barrier_sem, inc=1, device_id=nbr(ax),
                device_id_type=pl.DeviceIdType.MESH,
            )
        pl.semaphore_wait(barrier_sem, 3)

        recv_bufs = (rv0, rv1, rv2)
        stage_bufs = (st0, st1, st2)

        starts = [GROUPS[g][0] for g in range(3)]
        sizes = [GROUPS[g][1] for g in range(3)]
        rdmas = [None, None, None]

        for rnd in range(3):
            for g in range(3):
                ax = ORDERS[g][rnd]
                bit = m[ax]
                half = sizes[g] // 2
                send_start = starts[g] + (1 - bit) * half
                stage_bufs[g][0:half, :] = out_ref[
                    pl.ds(send_start, half), :
                ].astype(jnp.bfloat16)
                rdma = pltpu.make_async_remote_copy(
                    src_ref=stage_bufs[g].at[0:half],
                    dst_ref=recv_bufs[g].at[pl.ds(recv_off[g][rnd], half)],
                    send_sem=send_sems.at[g * 6 + rnd],
                    recv_sem=recv_sems.at[g * 6 + rnd],
                    device_id=nbr(ax),
                    device_id_type=pl.DeviceIdType.MESH,
                )
                rdma.start()
                rdmas[g] = rdma
                starts[g] = starts[g] + bit * half
                sizes[g] = half
            for g in range(3):
                rdmas[g].wait()
                out_ref[pl.ds(starts[g], sizes[g]), :] += recv_bufs[g][
                    pl.ds(recv_off[g][rnd], sizes[g]), :
                ].astype(jnp.float32)

        other_starts = [None, None, None]
        for j in range(3):
            rnd = 3 + j
            for g in range(3):
                ax = ORDERS[g][2 - j]
                bit = m[ax]
                sz = sizes[g]
                stage_bufs[g][0:sz, :] = out_ref[
                    pl.ds(starts[g], sz), :
                ].astype(jnp.bfloat16)
                rdma = pltpu.make_async_remote_copy(
                    src_ref=stage_bufs[g].at[0:sz],
                    dst_ref=recv_bufs[g].at[pl.ds(recv_off[g][rnd], sz)],
                    send_sem=send_sems.at[g * 6 + rnd],
                    recv_sem=recv_sems.at[g * 6 + rnd],
                    device_id=nbr(ax),
                    device_id_type=pl.DeviceIdType.MESH,
                )
                rdma.start()
                rdmas[g] = rdma
                other_starts[g] = starts[g] + (1 - 2 * bit) * sz
                starts[g] = starts[g] - bit * sz
            for g in range(3):
                sz = sizes[g]
                rdmas[g].wait()
                out_ref[pl.ds(other_starts[g], sz), :] = recv_bufs[g][
                    pl.ds(recv_off[g][rnd], sz), :
                ].astype(jnp.float32)
                sizes[g] = 2 * sz

    return pl.pallas_call(
        body,
        out_shape=jax.ShapeDtypeStruct((M, M), jnp.float32),
        in_specs=[
            pl.BlockSpec(memory_space=pltpu.VMEM),
            pl.BlockSpec(memory_space=pltpu.VMEM),
        ],
        out_specs=pl.BlockSpec(memory_space=pltpu.VMEM),
        scratch_shapes=[
            pltpu.VMEM((recv_rows[0], M), jnp.bfloat16),
            pltpu.VMEM((recv_rows[1], M), jnp.bfloat16),
            pltpu.VMEM((recv_rows[2], M), jnp.bfloat16),
            pltpu.VMEM((GROUPS[0][1] // 2, M), jnp.bfloat16),
            pltpu.VMEM((GROUPS[1][1] // 2, M), jnp.bfloat16),
            pltpu.VMEM((GROUPS[2][1] // 2, M), jnp.bfloat16),
            pltpu.SemaphoreType.DMA((18,)),
            pltpu.SemaphoreType.DMA((18,)),
        ],
        compiler_params=pltpu.CompilerParams(
            collective_id=0,
            vmem_limit_bytes=63 * 1024 * 1024,
        ),
    )(a, b)


# device time: 136626 ns/iter; 2.8464x vs baseline; 1.0041x over previous
import jax
import jax.numpy as jnp
from jax import lax
from jax.experimental import pallas as pl
from jax.experimental.pallas import tpu as pltpu

M = 2048
CHUNK = 2048

AXES = ("x", "y", "z")
GROUPS = ((0, 768), (768, 640), (1408, 640))
ORDERS = ((0, 1, 2), (1, 2, 0), (2, 0, 1))


def kernel(dy, W):
    r = lax.axis_index("y") * 2 + lax.axis_index("z")
    a = lax.dynamic_slice_in_dim(dy, r * CHUNK, CHUNK, axis=1).astype(jnp.bfloat16)
    b = lax.dynamic_slice_in_dim(W, r * CHUNK, CHUNK, axis=1).astype(jnp.bfloat16)

    recv_off = []
    recv_rows = []
    for _, rows in GROUPS:
        sizes = [rows // 2, rows // 4, rows // 8, rows // 8, rows // 4, rows // 2]
        offs = []
        o = 0
        for s in sizes:
            offs.append(o)
            o += s
        recv_off.append(offs)
        recv_rows.append(o)

    def body(a_ref, b_ref, out_ref, rv0, rv1, rv2, st0, st1, st2,
             send_sems, recv_sems):
        m = [lax.axis_index(ax) for ax in AXES]

        def nbr(ax):
            t = list(m)
            t[ax] = 1 - t[ax]
            return tuple(t)

        barrier_sem = pltpu.get_barrier_semaphore()
        for ax in range(3):
            pl.semaphore_signal(
                barrier_sem, inc=1, device_id=nbr(ax),
                device_id_type=pl.DeviceIdType.MESH,
            )
        pl.semaphore_wait(barrier_sem, 3)

        recv_bufs = (rv0, rv1, rv2)
        stage_bufs = (st0, st1, st2)

        starts = [GROUPS[g][0] for g in range(3)]
        sizes = [GROUPS[g][1] for g in range(3)]
        rdmas = [None, None, None]

        for g in range(3):
            gs, gr = GROUPS[g]
            half = gr // 2
            for s in range(2):
                out_ref[gs + s * half:gs + (s + 1) * half, :] = lax.dot_general(
                    a_ref[gs + s * half:gs + (s + 1) * half, :],
                    b_ref[...],
                    dimension_numbers=(((1,), (1,)), ((), ())),
                    preferred_element_type=jnp.float32,
                )
            ax = ORDERS[g][0]
            bit = m[ax]
            send_start = gs + (1 - bit) * half
            stage_bufs[g][0:half, :] = out_ref[
                pl.ds(send_start, half), :
            ].astype(jnp.bfloat16)
            rdma = pltpu.make_async_remote_copy(
                src_ref=stage_bufs[g].at[0:half],
                dst_ref=recv_bufs[g].at[pl.ds(recv_off[g][0], half)],
                send_sem=send_sems.at[g * 6],
                recv_sem=recv_sems.at[g * 6],
                device_id=nbr(ax),
                device_id_type=pl.DeviceIdType.MESH,
            )
            rdma.start()
            rdmas[g] = rdma
            starts[g] = gs + bit * half
            sizes[g] = half
        for g in range(3):
            rdmas[g].wait()
            out_ref[pl.ds(starts[g], sizes[g]), :] += recv_bufs[g][
                pl.ds(recv_off[g][0], sizes[g]), :
            ].astype(jnp.float32)

        for rnd in range(1, 3):
            for g in range(3):
                ax = ORDERS[g][rnd]
                bit = m[ax]
                half = sizes[g] // 2
                send_start = starts[g] + (1 - bit) * half
                stage_bufs[g][0:half, :] = out_ref[
                    pl.ds(send_start, half), :
                ].astype(jnp.bfloat16)
                rdma = pltpu.make_async_remote_copy(
                    src_ref=stage_bufs[g].at[0:half],
                    dst_ref=recv_bufs[g].at[pl.ds(recv_off[g][rnd], half)],
                    send_sem=send_sems.at[g * 6 + rnd],
                    recv_sem=recv_sems.at[g * 6 + rnd],
                    device_id=nbr(ax),
                    device_id_type=pl.DeviceIdType.MESH,
                )
                rdma.start()
                rdmas[g] = rdma
                starts[g] = starts[g] + bit * half
                sizes[g] = half
            for g in range(3):
                rdmas[g].wait()
                out_ref[pl.ds(starts[g], sizes[g]), :] += recv_bufs[g][
                    pl.ds(recv_off[g][rnd], sizes[g]), :
                ].astype(jnp.float32)

        other_starts = [None, None, None]
        for j in range(3):
            rnd = 3 + j
            for g in range(3):
                ax = ORDERS[g][2 - j]
                bit = m[ax]
                sz = sizes[g]
                stage_bufs[g][0:sz, :] = out_ref[
                    pl.ds(starts[g], sz), :
                ].astype(jnp.bfloat16)
                rdma = pltpu.make_async_remote_copy(
                    src_ref=stage_bufs[g].at[0:sz],
                    dst_ref=recv_bufs[g].at[pl.ds(recv_off[g][rnd], sz)],
                    send_sem=send_sems.at[g * 6 + rnd],
                    recv_sem=recv_sems.at[g * 6 + rnd],
                    device_id=nbr(ax),
                    device_id_type=pl.DeviceIdType.MESH,
                )
                rdma.start()
                rdmas[g] = rdma
                other_starts[g] = starts[g] + (1 - 2 * bit) * sz
                starts[g] = starts[g] - bit * sz
            for g in range(3):
                sz = sizes[g]
                rdmas[g].wait()
                out_ref[pl.ds(other_starts[g], sz), :] = recv_bufs[g][
                    pl.ds(recv_off[g][rnd], sz), :
                ].astype(jnp.float32)
                sizes[g] = 2 * sz

    return pl.pallas_call(
        body,
        out_shape=jax.ShapeDtypeStruct((M, M), jnp.float32),
        in_specs=[
            pl.BlockSpec(memory_space=pltpu.VMEM),
            pl.BlockSpec(memory_space=pltpu.VMEM),
        ],
        out_specs=pl.BlockSpec(memory_space=pltpu.VMEM),
        scratch_shapes=[
            pltpu.VMEM((recv_rows[0], M), jnp.bfloat16),
            pltpu.VMEM((recv_rows[1], M), jnp.bfloat16),
            pltpu.VMEM((recv_rows[2], M), jnp.bfloat16),
            pltpu.VMEM((GROUPS[0][1] // 2, M), jnp.bfloat16),
            pltpu.VMEM((GROUPS[1][1] // 2, M), jnp.bfloat16),
            pltpu.VMEM((GROUPS[2][1] // 2, M), jnp.bfloat16),
            pltpu.SemaphoreType.DMA((18,)),
            pltpu.SemaphoreType.DMA((18,)),
        ],
        compiler_params=pltpu.CompilerParams(
            collective_id=0,
            vmem_limit_bytes=63 * 1024 * 1024,
        ),
    )(a, b)
